# baseline (speedup 1.0000x reference)
import jax
import jax.numpy as jnp
from jax.experimental import pallas as pl
from jax.experimental.pallas import tpu as pltpu
from jax.experimental.pallas import tpu_sc as plsc

B = 256
D = 2048
S = 49
C = 1000
NSTREAM = 7
NSTEP = 2 * S // NSTREAM
HALF = B // 2


def _argmax_body(lgt_ref, cls_ref, lgc_ref):
    lgt = lgt_ref[...]
    lgc_ref[...] = lgt
    row = jax.lax.broadcasted_iota(jnp.int32, (C, B), 0)
    col_max = jnp.max(lgt, axis=0, keepdims=True)
    cls_ref[...] = jnp.min(jnp.where(lgt == col_max, row, C), axis=0,
                           keepdims=True)


def _counts_sc_body(cls_ref, out_ref, idx_s, cnt_s, sem):
    @pl.when(jax.lax.axis_index("c") == 0)
    def _():
        pltpu.async_copy(cls_ref.at[0], idx_s, sem).wait()

        @pl.loop(0, C)
        def _zero(i):
            cnt_s[i] = 0.0

        @pl.loop(0, B)
        def _count(i):
            cnt_s[idx_s[i]] += 1.0

        pltpu.async_copy(cnt_s, out_ref, sem).wait()


def _mean_body(*refs):
    x_refs = refs[:NSTREAM]
    xm_ref, xc_ref = refs[NSTREAM:]
    i = pl.program_id(0)

    def _tree(ps):
        while len(ps) > 1:
            ps = [a + b for a, b in zip(ps[::2], ps[1::2])] + (
                ps[-1:] if len(ps) % 2 else [])
        return ps[0]

    a = _tree([r[0] for r in x_refs[0::2]])
    b = _tree([r[0] for r in x_refs[1::2]])

    for k, r in enumerate(x_refs):
        xc_ref[k] = r[0]

    even = i % 2 == 0

    @pl.when(i == 0)
    def _init():
        xm_ref[:HALF, :] = a
        xm_ref[HALF:, :] = b

    @pl.when((i > 0) & even)
    def _acc_even():
        xm_ref[:HALF, :] += a
        xm_ref[HALF:, :] += b

    @pl.when(jnp.logical_not(even))
    def _acc_odd():
        xm_ref[:HALF, :] += b
        xm_ref[HALF:, :] += a

    @pl.when(i == NSTEP - 1)
    def _scale():
        xm_ref[...] *= (1.0 / S)


def _merge_body(xm_ref, cls_ref, cnt_ref, pt_ref, out_ref):
    xm = xm_ref[...]
    row = jax.lax.broadcasted_iota(jnp.int32, (C, B), 0)
    onehot = (row == cls_ref[...]).astype(jnp.bfloat16)
    counts = cnt_ref[...]
    sums = jax.lax.dot_general(
        onehot, xm.astype(jnp.bfloat16),
        dimension_numbers=(((1,), (0,)), ((), ())),
        preferred_element_type=jnp.float32,
    )

    mean = sums / jnp.maximum(counts, 1.0)
    pt = pt_ref[...]
    dot = jnp.sum(pt * mean, axis=1, keepdims=True)
    denom = jnp.maximum(
        jnp.sqrt(jnp.sum(pt * pt, axis=1, keepdims=True))
        * jnp.sqrt(jnp.sum(mean * mean, axis=1, keepdims=True)),
        1e-8,
    )
    mom = dot / denom
    exist = counts > 0.0
    out_ref[...] = jnp.where(exist, pt * mom + mean * (1.0 - mom), pt)


def _x_spec(j):
    return pl.BlockSpec((1, HALF, D), lambda i, j=j: (NSTREAM * i + j, 0, 0))


@jax.jit
def _run(x, class_logits, prototypes):
    xt = jax.lax.transpose(x, (2, 3, 0, 1)).reshape(2 * S, HALF, D)
    lgt = jax.lax.transpose(class_logits, (1, 0))

    cls, lg_copy = pl.pallas_call(
        _argmax_body,
        out_shape=(
            jax.ShapeDtypeStruct((1, B), jnp.int32),
            jax.ShapeDtypeStruct((C, B), jnp.float32),
        ),
    )(lgt)
    lg_out = jax.lax.transpose(lg_copy, (1, 0))

    counts_kernel = pl.kernel(
        _counts_sc_body,
        out_type=jax.ShapeDtypeStruct((C,), jnp.float32),
        mesh=plsc.ScalarSubcoreMesh(axis_name="c", num_cores=2),
        scratch_types=[
            pltpu.SMEM((B,), jnp.int32),
            pltpu.SMEM((C,), jnp.float32),
            pltpu.SemaphoreType.DMA,
        ],
    )
    counts = counts_kernel(cls)

    x_mapped, x_copy = pl.pallas_call(
        _mean_body,
        grid=(NSTEP,),
        in_specs=[_x_spec(j) for j in range(NSTREAM)],
        out_specs=(
            pl.BlockSpec((B, D), lambda i: (0, 0)),
            pl.BlockSpec((NSTREAM, HALF, D), lambda i: (i, 0, 0)),
        ),
        out_shape=(
            jax.ShapeDtypeStruct((B, D), jnp.float32),
            jax.ShapeDtypeStruct((2 * S, HALF, D), jnp.float32),
        ),
    )(*([xt] * NSTREAM))
    x_out = jax.lax.transpose(x_copy.reshape(7, 7, B, D), (2, 3, 0, 1))

    new_prototypes = pl.pallas_call(
        _merge_body,
        out_shape=jax.ShapeDtypeStruct((C, D), jnp.float32),
    )(x_mapped, cls, counts.reshape(C, 1), prototypes)
    return new_prototypes, cls.reshape(B), x_mapped, x_out, lg_out


def kernel(x, class_logits, prototypes, step, thresholds):
    new_prototypes, max_cls, x_mapped, x_out, lg_out = _run(
        x, class_logits, prototypes)
    return (new_prototypes, step, x_out, lg_out, max_cls, x_mapped)


# speedup vs baseline: 1.2172x; 1.2172x over previous
import jax
import jax.numpy as jnp
from jax.experimental import pallas as pl
from jax.experimental.pallas import tpu as pltpu
from jax.experimental.pallas import tpu_sc as plsc

B = 256
D = 2048
S = 49
C = 1000
NSTREAM = 7
NSTEP = 2 * S // NSTREAM
HALF = B // 2


def _argmax_body(lgt_ref, cls_ref, lgc_ref):
    lgt = lgt_ref[...]
    lgc_ref[...] = lgt
    row = jax.lax.broadcasted_iota(jnp.int32, (C, B), 0)
    col_max = jnp.max(lgt, axis=0, keepdims=True)
    cls_ref[...] = jnp.min(jnp.where(lgt == col_max, row, C), axis=0,
                           keepdims=True)


def _counts_sc_body(cls_ref, out_ref, idx_s, cnt_s, sem):
    @pl.when(jax.lax.axis_index("c") == 0)
    def _():
        pltpu.async_copy(cls_ref.at[0], idx_s, sem).wait()

        @pl.loop(0, C)
        def _zero(i):
            cnt_s[i] = 0.0

        @pl.loop(0, B)
        def _count(i):
            cnt_s[idx_s[i]] += 1.0

        pltpu.async_copy(cnt_s, out_ref, sem).wait()


def _mean_body(*refs):
    x_refs = refs[:NSTREAM]
    xm_ref, xc_ref = refs[NSTREAM:]
    i = pl.program_id(0)

    def _tree(ps):
        while len(ps) > 1:
            ps = [a + b for a, b in zip(ps[::2], ps[1::2])] + (
                ps[-1:] if len(ps) % 2 else [])
        return ps[0]

    a = _tree([r[0] for r in x_refs[0::2]])
    b = _tree([r[0] for r in x_refs[1::2]])

    for k, r in enumerate(x_refs):
        xc_ref[k] = r[0]

    even = i % 2 == 0

    @pl.when(i == 0)
    def _init():
        xm_ref[:HALF, :] = a
        xm_ref[HALF:, :] = b

    @pl.when((i > 0) & even)
    def _acc_even():
        xm_ref[:HALF, :] += a
        xm_ref[HALF:, :] += b

    @pl.when(jnp.logical_not(even))
    def _acc_odd():
        xm_ref[:HALF, :] += b
        xm_ref[HALF:, :] += a

    @pl.when(i == NSTEP - 1)
    def _scale():
        xm_ref[...] *= (1.0 / S)


def _merge_body(xm_ref, cls_ref, pt_ref, out_ref):
    xm = xm_ref[...]
    row = jax.lax.broadcasted_iota(jnp.int32, (C, B), 0)
    onehot = (row == cls_ref[...]).astype(jnp.bfloat16)
    counts = jnp.sum(onehot.astype(jnp.float32), axis=1, keepdims=True)
    sums = jax.lax.dot_general(
        onehot, xm.astype(jnp.bfloat16),
        dimension_numbers=(((1,), (0,)), ((), ())),
        preferred_element_type=jnp.float32,
    )

    mean = sums / jnp.maximum(counts, 1.0)
    pt = pt_ref[...]
    dot = jnp.sum(pt * mean, axis=1, keepdims=True)
    denom = jnp.maximum(
        jnp.sqrt(jnp.sum(pt * pt, axis=1, keepdims=True))
        * jnp.sqrt(jnp.sum(mean * mean, axis=1, keepdims=True)),
        1e-8,
    )
    mom = dot / denom
    exist = counts > 0.0
    out_ref[...] = jnp.where(exist, pt * mom + mean * (1.0 - mom), pt)


def _x_spec(j):
    return pl.BlockSpec((1, HALF, D), lambda i, j=j: (NSTREAM * i + j, 0, 0))


@jax.jit
def _run(x, class_logits, prototypes):
    xt = jax.lax.transpose(x, (2, 3, 0, 1)).reshape(2 * S, HALF, D)
    lgt = jax.lax.transpose(class_logits, (1, 0))

    cls, lg_copy = pl.pallas_call(
        _argmax_body,
        out_shape=(
            jax.ShapeDtypeStruct((1, B), jnp.int32),
            jax.ShapeDtypeStruct((C, B), jnp.float32),
        ),
    )(lgt)
    lg_out = jax.lax.transpose(lg_copy, (1, 0))

    x_mapped, x_copy = pl.pallas_call(
        _mean_body,
        grid=(NSTEP,),
        in_specs=[_x_spec(j) for j in range(NSTREAM)],
        out_specs=(
            pl.BlockSpec((B, D), lambda i: (0, 0)),
            pl.BlockSpec((NSTREAM, HALF, D), lambda i: (i, 0, 0)),
        ),
        out_shape=(
            jax.ShapeDtypeStruct((B, D), jnp.float32),
            jax.ShapeDtypeStruct((2 * S, HALF, D), jnp.float32),
        ),
    )(*([xt] * NSTREAM))
    x_out = jax.lax.transpose(x_copy.reshape(7, 7, B, D), (2, 3, 0, 1))

    new_prototypes = pl.pallas_call(
        _merge_body,
        out_shape=jax.ShapeDtypeStruct((C, D), jnp.float32),
    )(x_mapped, cls, prototypes)
    return new_prototypes, cls.reshape(B), x_mapped, x_out, lg_out


def kernel(x, class_logits, prototypes, step, thresholds):
    new_prototypes, max_cls, x_mapped, x_out, lg_out = _run(
        x, class_logits, prototypes)
    return (new_prototypes, step, x_out, lg_out, max_cls, x_mapped)


# speedup vs baseline: 1.2388x; 1.0177x over previous
import jax
import jax.numpy as jnp
from jax.experimental import pallas as pl

B = 256
D = 2048
S = 49
C = 1000
NSTREAM = 7
NSTEP = 2 * S // NSTREAM
HALF = B // 2


def _mean_body(*refs):
    x_refs = refs[:NSTREAM]
    xm_ref, xc_ref = refs[NSTREAM:]
    i = pl.program_id(0)

    def _tree(ps):
        while len(ps) > 1:
            ps = [a + b for a, b in zip(ps[::2], ps[1::2])] + (
                ps[-1:] if len(ps) % 2 else [])
        return ps[0]

    a = _tree([r[0] for r in x_refs[0::2]])
    b = _tree([r[0] for r in x_refs[1::2]])

    for k, r in enumerate(x_refs):
        xc_ref[k] = r[0]

    even = i % 2 == 0

    @pl.when(i == 0)
    def _init():
        xm_ref[:HALF, :] = a
        xm_ref[HALF:, :] = b

    @pl.when((i > 0) & even)
    def _acc_even():
        xm_ref[:HALF, :] += a
        xm_ref[HALF:, :] += b

    @pl.when(jnp.logical_not(even))
    def _acc_odd():
        xm_ref[:HALF, :] += b
        xm_ref[HALF:, :] += a

    @pl.when(i == NSTEP - 1)
    def _scale():
        xm_ref[...] *= (1.0 / S)


def _merge_body(xm_ref, lgt_ref, pt_ref, out_ref, cls_ref, lgc_ref):
    xm = xm_ref[...]
    lgt = lgt_ref[...]
    lgc_ref[...] = lgt
    row = jax.lax.broadcasted_iota(jnp.int32, (C, B), 0)
    col_max = jnp.max(lgt, axis=0, keepdims=True)
    first = jnp.min(jnp.where(lgt == col_max, row, C), axis=0, keepdims=True)
    cls_ref[...] = first

    onehot = (row == first).astype(jnp.float32)
    counts = jnp.sum(onehot, axis=1, keepdims=True)
    sums = jax.lax.dot_general(
        onehot.astype(jnp.bfloat16), xm.astype(jnp.bfloat16),
        dimension_numbers=(((1,), (0,)), ((), ())),
        preferred_element_type=jnp.float32,
    )

    mean = sums / jnp.maximum(counts, 1.0)
    pt = pt_ref[...]
    dot = jnp.sum(pt * mean, axis=1, keepdims=True)
    denom = jnp.maximum(
        jnp.sqrt(jnp.sum(pt * pt, axis=1, keepdims=True))
        * jnp.sqrt(jnp.sum(mean * mean, axis=1, keepdims=True)),
        1e-8,
    )
    mom = dot / denom
    exist = counts > 0.0
    out_ref[...] = jnp.where(exist, pt * mom + mean * (1.0 - mom), pt)


def _x_spec(j):
    return pl.BlockSpec((1, HALF, D), lambda i, j=j: (NSTREAM * i + j, 0, 0))


@jax.jit
def _run(x, class_logits, prototypes):
    xt = jax.lax.transpose(x, (2, 3, 0, 1)).reshape(2 * S, HALF, D)
    lgt = jax.lax.transpose(class_logits, (1, 0))

    x_mapped, x_copy = pl.pallas_call(
        _mean_body,
        grid=(NSTEP,),
        in_specs=[_x_spec(j) for j in range(NSTREAM)],
        out_specs=(
            pl.BlockSpec((B, D), lambda i: (0, 0)),
            pl.BlockSpec((NSTREAM, HALF, D), lambda i: (i, 0, 0)),
        ),
        out_shape=(
            jax.ShapeDtypeStruct((B, D), jnp.float32),
            jax.ShapeDtypeStruct((2 * S, HALF, D), jnp.float32),
        ),
    )(*([xt] * NSTREAM))
    x_out = jax.lax.transpose(x_copy.reshape(7, 7, B, D), (2, 3, 0, 1))

    new_prototypes, cls, lg_copy = pl.pallas_call(
        _merge_body,
        out_shape=(
            jax.ShapeDtypeStruct((C, D), jnp.float32),
            jax.ShapeDtypeStruct((1, B), jnp.int32),
            jax.ShapeDtypeStruct((C, B), jnp.float32),
        ),
    )(x_mapped, lgt, prototypes)
    lg_out = jax.lax.transpose(lg_copy, (1, 0))
    return new_prototypes, cls.reshape(B), x_mapped, x_out, lg_out


def kernel(x, class_logits, prototypes, step, thresholds):
    new_prototypes, max_cls, x_mapped, x_out, lg_out = _run(
        x, class_logits, prototypes)
    return (new_prototypes, step, x_out, lg_out, max_cls, x_mapped)
